# baseline (device time: 428894 ns/iter reference)
import jax
import jax.numpy as jnp
from jax import lax
from jax.experimental import pallas as pl
from jax.experimental.pallas import tpu as pltpu

B = 32
NB = 256
BS = 32
H = 16
D = 128
ZDIM = 4
CH = 8
CK = CH * BS


def kernel(Q, K, V, bt, lens):
    n_local_pages = K.shape[0]
    my_z = lax.axis_index("z")

    pos = jnp.arange(NB, dtype=jnp.int32)[None, :]
    owned = (bt // n_local_pages == my_z) & (pos < lens[:, None])
    dest = jnp.where(owned, jnp.cumsum(owned, axis=1) - 1, NB - 1)
    local = jnp.clip(bt - my_z * n_local_pages, 0, n_local_pages - 1)
    rows = jnp.broadcast_to(jnp.arange(B, dtype=jnp.int32)[:, None], (B, NB))
    cpt = jnp.zeros((B, NB), jnp.int32).at[rows, dest].set(
        jnp.where(owned, local, 0), mode="drop")
    counts = owned.sum(axis=1).astype(jnp.int32)

    scale = D ** -0.5

    def body(q_ref, k_ref, v_ref, cpt_ref, cnt_ref, out_ref,
             comm_ref, kbuf, vbuf, kv_sems, send_sems, recv_sems):
        my_x = lax.axis_index("x")
        my_y = lax.axis_index("y")
        mz = lax.axis_index("z")

        ones_col = jnp.ones((D, 1), jnp.bfloat16)

        def start_chunk(i, t, slot):
            for c in range(CH):
                p = cpt_ref[i, t * CH + c]
                pltpu.make_async_copy(
                    k_ref.at[p], kbuf.at[slot, c],
                    kv_sems.at[0, slot, c]).start()
                pltpu.make_async_copy(
                    v_ref.at[p], vbuf.at[slot, c],
                    kv_sems.at[1, slot, c]).start()

        def wait_chunk(slot):
            for c in range(CH):
                pltpu.make_async_copy(
                    k_ref.at[0], kbuf.at[slot, c],
                    kv_sems.at[0, slot, c]).wait()
                pltpu.make_async_copy(
                    v_ref.at[0], vbuf.at[slot, c],
                    kv_sems.at[1, slot, c]).wait()

        def batch_body(i, carry):
            T = cnt_ref[i]
            nch = lax.div(T + CH - 1, CH)
            qb = q_ref[i, 0] * scale

            @pl.when(nch > 0)
            def _():
                start_chunk(i, 0, 0)

            def chunk_body(t, mla):
                m, l, acc = mla
                slot = lax.rem(t, 2)

                @pl.when(t + 1 < nch)
                def _():
                    start_chunk(i, t + 1, lax.rem(t + 1, 2))

                wait_chunk(slot)
                kc = kbuf[slot].reshape(CK, H, D)
                vc = vbuf[slot].reshape(CK, H, D)
                m1 = (kc * qb[None]).astype(jnp.bfloat16).reshape(CK * H, D)
                s2 = jax.lax.dot_general(
                    m1, ones_col, (((1,), (0,)), ((), ())),
                    preferred_element_type=jnp.float32)
                s = s2.reshape(CK, H, 1)
                entry = t * CH + \
                    lax.broadcasted_iota(jnp.int32, (CK, H, 1), 0) // BS
                s = jnp.where(entry < T, s, -1e30)
                m_new = jnp.maximum(m, jnp.max(s, axis=0))
                alpha = jnp.exp(m - m_new)
                p = jnp.exp(s - m_new[None])
                l_new = l * alpha + jnp.sum(p, axis=0)
                pv = jnp.sum(jnp.broadcast_to(p, (CK, H, D)) * vc, axis=0)
                acc_new = acc * alpha + pv
                return m_new, l_new, acc_new

            init = (jnp.full((H, 1), -1e30, jnp.float32),
                    jnp.zeros((H, 1), jnp.float32),
                    jnp.zeros((H, D), jnp.float32))
            m, l, acc = lax.fori_loop(0, nch, chunk_body, init)

            comm_ref[0, 0, i] = acc
            comm_ref[0, 1, i] = jnp.broadcast_to(m, (H, D))
            comm_ref[0, 2, i] = jnp.broadcast_to(l, (H, D))
            return carry

        lax.fori_loop(0, B, batch_body, 0)

        bsem = pltpu.get_barrier_semaphore()
        for dz in (1, 2, 3):
            pl.semaphore_signal(
                bsem, inc=1,
                device_id=(my_x, my_y, lax.rem(mz + dz, ZDIM)),
                device_id_type=pl.DeviceIdType.MESH,
            )
        pl.semaphore_wait(bsem, 3)

        sends = []
        for dz in (1, 2, 3):
            rdma = pltpu.make_async_remote_copy(
                src_ref=comm_ref.at[0],
                dst_ref=comm_ref.at[ZDIM - dz],
                send_sem=send_sems.at[dz],
                recv_sem=recv_sems.at[ZDIM - dz],
                device_id=(my_x, my_y, lax.rem(mz + dz, ZDIM)),
                device_id_type=pl.DeviceIdType.MESH,
            )
            rdma.start()
            sends.append(rdma)

        for s in (1, 2, 3):
            recv = pltpu.make_async_remote_copy(
                src_ref=comm_ref.at[s],
                dst_ref=comm_ref.at[s],
                send_sem=send_sems.at[0],
                recv_sem=recv_sems.at[s],
                device_id=(my_x, my_y, mz),
                device_id_type=pl.DeviceIdType.MESH,
            )
            recv.wait_recv()
        for rdma in sends:
            rdma.wait_send()

        mall = comm_ref[:, 1]
        mmax = jnp.max(mall, axis=0)
        w = jnp.exp(mall - mmax[None])
        o = jnp.sum(comm_ref[:, 0] * w, axis=0)
        lsum = jnp.sum(comm_ref[:, 2] * w, axis=0)
        out_ref[:, 0] = o / lsum

    return pl.pallas_call(
        body,
        out_shape=jax.ShapeDtypeStruct((B, 1, H, D), jnp.float32),
        in_specs=[
            pl.BlockSpec(memory_space=pltpu.VMEM),
            pl.BlockSpec(memory_space=pl.ANY),
            pl.BlockSpec(memory_space=pl.ANY),
            pl.BlockSpec(memory_space=pltpu.SMEM),
            pl.BlockSpec(memory_space=pltpu.SMEM),
        ],
        out_specs=pl.BlockSpec(memory_space=pltpu.VMEM),
        scratch_shapes=[
            pltpu.VMEM((ZDIM, 3, B, H, D), jnp.float32),
            pltpu.VMEM((2, CH, BS, H, D), jnp.float32),
            pltpu.VMEM((2, CH, BS, H, D), jnp.float32),
            pltpu.SemaphoreType.DMA((2, 2, CH)),
            pltpu.SemaphoreType.DMA((4,)),
            pltpu.SemaphoreType.DMA((4,)),
        ],
        compiler_params=pltpu.CompilerParams(collective_id=0),
    )(Q, K, V, cpt, counts)


# device time: 419545 ns/iter; 1.0223x vs baseline; 1.0223x over previous
import jax
import jax.numpy as jnp
from jax import lax
from jax.experimental import pallas as pl
from jax.experimental.pallas import tpu as pltpu

B = 32
NB = 256
BS = 32
H = 16
D = 128
ZDIM = 4
CH = 8
CK = CH * BS
GMAX = B * (NB // CH)


def kernel(Q, K, V, bt, lens):
    n_local_pages = K.shape[0]
    my_z = lax.axis_index("z")

    pos = jnp.arange(NB, dtype=jnp.int32)[None, :]
    owned = (bt // n_local_pages == my_z) & (pos < lens[:, None])
    dest = jnp.where(owned, jnp.cumsum(owned, axis=1) - 1, NB - 1)
    local = jnp.clip(bt - my_z * n_local_pages, 0, n_local_pages - 1)
    rows = jnp.broadcast_to(jnp.arange(B, dtype=jnp.int32)[:, None], (B, NB))
    cpt = jnp.zeros((B, NB), jnp.int32).at[rows, dest].set(
        jnp.where(owned, local, 0), mode="drop")
    counts = owned.sum(axis=1).astype(jnp.int32)

    chunks = (counts + CH - 1) // CH
    starts = jnp.cumsum(chunks) - chunks
    g_idx = jnp.arange(GMAX, dtype=jnp.int32)
    n_chunks = jnp.sum(chunks).astype(jnp.int32)
    sched_b = jnp.sum(
        (g_idx[:, None] >= (starts + chunks)[None, :]).astype(jnp.int32),
        axis=1)
    sched_b = jnp.minimum(sched_b, B - 1)
    sched_t = g_idx - jnp.take(starts, sched_b)
    sched_fin = (sched_t == jnp.take(chunks, sched_b) - 1).astype(jnp.int32)

    scale = D ** -0.5

    def body(q_ref, k_ref, v_ref, cpt_ref, cnt_ref, sb_ref, st_ref, fin_ref,
             ng_ref, out_ref, comm_ref, kbuf, vbuf,
             kv_sems, send_sems, recv_sems):
        my_x = lax.axis_index("x")
        my_y = lax.axis_index("y")
        mz = lax.axis_index("z")

        ones_col = jnp.ones((D, 1), jnp.bfloat16)
        G = ng_ref[0]

        comm_ref[0, 0] = jnp.zeros((B, H, D), jnp.float32)
        comm_ref[0, 1] = jnp.full((B, H, D), -1e30, jnp.float32)
        comm_ref[0, 2] = jnp.zeros((B, H, D), jnp.float32)

        def start_chunk(g, slot):
            b = sb_ref[g]
            t = st_ref[g]
            for c in range(CH):
                p = cpt_ref[b, t * CH + c]
                pltpu.make_async_copy(
                    k_ref.at[p], kbuf.at[slot, c],
                    kv_sems.at[0, slot, c]).start()
                pltpu.make_async_copy(
                    v_ref.at[p], vbuf.at[slot, c],
                    kv_sems.at[1, slot, c]).start()

        def wait_chunk(slot):
            for c in range(CH):
                pltpu.make_async_copy(
                    k_ref.at[0], kbuf.at[slot, c],
                    kv_sems.at[0, slot, c]).wait()
                pltpu.make_async_copy(
                    v_ref.at[0], vbuf.at[slot, c],
                    kv_sems.at[1, slot, c]).wait()

        @pl.when(G > 0)
        def _():
            start_chunk(0, 0)

        def chunk_body(g, mla):
            m, l, acc = mla
            b = sb_ref[g]
            t = st_ref[g]
            T = cnt_ref[b]
            slot = lax.rem(g, 2)

            @pl.when(g + 1 < G)
            def _():
                start_chunk(g + 1, lax.rem(g + 1, 2))

            wait_chunk(slot)
            qb = q_ref[b, 0] * scale
            kc = kbuf[slot].reshape(CK, H, D)
            vc = vbuf[slot].reshape(CK, H, D)
            m1 = (kc * qb[None]).astype(jnp.bfloat16).reshape(CK * H, D)
            s2 = jax.lax.dot_general(
                m1, ones_col, (((1,), (0,)), ((), ())),
                preferred_element_type=jnp.float32)
            s = s2.reshape(CK, H, 1)
            entry = t * CH + \
                lax.broadcasted_iota(jnp.int32, (CK, H, 1), 0) // BS
            s = jnp.where(entry < T, s, -1e30)
            m_new = jnp.maximum(m, jnp.max(s, axis=0))
            alpha = jnp.exp(m - m_new)
            p = jnp.exp(s - m_new[None])
            l_new = l * alpha + jnp.sum(p, axis=0)
            pv = jnp.sum(jnp.broadcast_to(p, (CK, H, D)) * vc, axis=0)
            acc_new = acc * alpha + pv

            fin = fin_ref[g]

            @pl.when(fin == 1)
            def _():
                comm_ref[0, 0, b] = acc_new
                comm_ref[0, 1, b] = jnp.broadcast_to(m_new, (H, D))
                comm_ref[0, 2, b] = jnp.broadcast_to(l_new, (H, D))

            keep = (fin == 0)
            return (jnp.where(keep, m_new, -1e30),
                    jnp.where(keep, l_new, 0.0),
                    jnp.where(keep, acc_new, 0.0))

        init = (jnp.full((H, 1), -1e30, jnp.float32),
                jnp.zeros((H, 1), jnp.float32),
                jnp.zeros((H, D), jnp.float32))
        lax.fori_loop(0, G, chunk_body, init)

        bsem = pltpu.get_barrier_semaphore()
        for dz in (1, 2, 3):
            pl.semaphore_signal(
                bsem, inc=1,
                device_id=(my_x, my_y, lax.rem(mz + dz, ZDIM)),
                device_id_type=pl.DeviceIdType.MESH,
            )
        pl.semaphore_wait(bsem, 3)

        sends = []
        for dz in (1, 2, 3):
            rdma = pltpu.make_async_remote_copy(
                src_ref=comm_ref.at[0],
                dst_ref=comm_ref.at[ZDIM - dz],
                send_sem=send_sems.at[dz],
                recv_sem=recv_sems.at[ZDIM - dz],
                device_id=(my_x, my_y, lax.rem(mz + dz, ZDIM)),
                device_id_type=pl.DeviceIdType.MESH,
            )
            rdma.start()
            sends.append(rdma)

        for s in (1, 2, 3):
            recv = pltpu.make_async_remote_copy(
                src_ref=comm_ref.at[s],
                dst_ref=comm_ref.at[s],
                send_sem=send_sems.at[0],
                recv_sem=recv_sems.at[s],
                device_id=(my_x, my_y, mz),
                device_id_type=pl.DeviceIdType.MESH,
            )
            recv.wait_recv()
        for rdma in sends:
            rdma.wait_send()

        mall = comm_ref[:, 1]
        mmax = jnp.max(mall, axis=0)
        w = jnp.exp(mall - mmax[None])
        o = jnp.sum(comm_ref[:, 0] * w, axis=0)
        lsum = jnp.sum(comm_ref[:, 2] * w, axis=0)
        out_ref[:, 0] = o / lsum

    return pl.pallas_call(
        body,
        out_shape=jax.ShapeDtypeStruct((B, 1, H, D), jnp.float32),
        in_specs=[
            pl.BlockSpec(memory_space=pltpu.VMEM),
            pl.BlockSpec(memory_space=pl.ANY),
            pl.BlockSpec(memory_space=pl.ANY),
            pl.BlockSpec(memory_space=pltpu.SMEM),
            pl.BlockSpec(memory_space=pltpu.SMEM),
            pl.BlockSpec(memory_space=pltpu.SMEM),
            pl.BlockSpec(memory_space=pltpu.SMEM),
            pl.BlockSpec(memory_space=pltpu.SMEM),
            pl.BlockSpec(memory_space=pltpu.SMEM),
        ],
        out_specs=pl.BlockSpec(memory_space=pltpu.VMEM),
        scratch_shapes=[
            pltpu.VMEM((ZDIM, 3, B, H, D), jnp.float32),
            pltpu.VMEM((2, CH, BS, H, D), jnp.float32),
            pltpu.VMEM((2, CH, BS, H, D), jnp.float32),
            pltpu.SemaphoreType.DMA((2, 2, CH)),
            pltpu.SemaphoreType.DMA((4,)),
            pltpu.SemaphoreType.DMA((4,)),
        ],
        compiler_params=pltpu.CompilerParams(collective_id=0),
    )(Q, K, V, cpt, counts, sched_b, sched_t, sched_fin,
      n_chunks.reshape(1))


# device time: 376842 ns/iter; 1.1381x vs baseline; 1.1133x over previous
import jax
import jax.numpy as jnp
from jax import lax
from jax.experimental import pallas as pl
from jax.experimental.pallas import tpu as pltpu

B = 32
NB = 256
BS = 32
H = 16
D = 128
ZDIM = 4
CH = 8
CK = CH * BS
GMAX = B * (NB // CH)


def kernel(Q, K, V, bt, lens):
    n_local_pages = K.shape[0]
    my_z = lax.axis_index("z")

    pos = jnp.arange(NB, dtype=jnp.int32)[None, :]
    owned = (bt // n_local_pages == my_z) & (pos < lens[:, None])
    dest = jnp.where(owned, jnp.cumsum(owned, axis=1) - 1, NB - 1)
    local = jnp.clip(bt - my_z * n_local_pages, 0, n_local_pages - 1)
    rows = jnp.broadcast_to(jnp.arange(B, dtype=jnp.int32)[:, None], (B, NB))
    cpt = jnp.zeros((B, NB), jnp.int32).at[rows, dest].set(
        jnp.where(owned, local, 0), mode="drop")
    counts = owned.sum(axis=1).astype(jnp.int32)

    scale = D ** -0.5

    def body(q_ref, k_ref, v_ref, cpt_ref, cnt_ref, out_ref,
             comm_ref, kbuf, vbuf, sb_ref, st_ref, fin_ref,
             kv_sems, send_sems, recv_sems):
        my_x = lax.axis_index("x")
        my_y = lax.axis_index("y")
        mz = lax.axis_index("z")

        ones_col = jnp.ones((D, 1), jnp.bfloat16)

        def build_batch(b, g):
            nch = lax.div(cnt_ref[b] + CH - 1, CH)

            def build_chunk(t, g2):
                sb_ref[g2] = b
                st_ref[g2] = t
                fin_ref[g2] = jnp.where(t == nch - 1, 1, 0)
                return g2 + 1

            return lax.fori_loop(0, nch, build_chunk, g)

        G = lax.fori_loop(0, B, build_batch, jnp.int32(0))

        comm_ref[0, 0] = jnp.zeros((B, H, D), jnp.float32)
        comm_ref[0, 1] = jnp.full((B, H, D), -1e30, jnp.float32)
        comm_ref[0, 2] = jnp.zeros((B, H, D), jnp.float32)

        def start_chunk(g, slot):
            b = sb_ref[g]
            t = st_ref[g]
            for c in range(CH):
                p = cpt_ref[b, t * CH + c]
                pltpu.make_async_copy(
                    k_ref.at[p], kbuf.at[slot, c],
                    kv_sems.at[0, slot, c]).start()
                pltpu.make_async_copy(
                    v_ref.at[p], vbuf.at[slot, c],
                    kv_sems.at[1, slot, c]).start()

        def wait_chunk(slot):
            for c in range(CH):
                pltpu.make_async_copy(
                    k_ref.at[0], kbuf.at[slot, c],
                    kv_sems.at[0, slot, c]).wait()
                pltpu.make_async_copy(
                    v_ref.at[0], vbuf.at[slot, c],
                    kv_sems.at[1, slot, c]).wait()

        @pl.when(G > 0)
        def _():
            start_chunk(0, 0)

        def chunk_body(g, mla):
            m, l, acc = mla
            b = sb_ref[g]
            t = st_ref[g]
            T = cnt_ref[b]
            slot = lax.rem(g, 2)

            @pl.when(g + 1 < G)
            def _():
                start_chunk(g + 1, lax.rem(g + 1, 2))

            wait_chunk(slot)
            qb = q_ref[b, 0] * scale
            kc = kbuf[slot].reshape(CK, H, D)
            vc = vbuf[slot].reshape(CK, H, D)
            m1 = (kc * qb[None]).astype(jnp.bfloat16).reshape(CK * H, D)
            s2 = jax.lax.dot_general(
                m1, ones_col, (((1,), (0,)), ((), ())),
                preferred_element_type=jnp.float32)
            s = s2.reshape(CK, H, 1)
            entry = t * CH + \
                lax.broadcasted_iota(jnp.int32, (CK, H, 1), 0) // BS
            s = jnp.where(entry < T, s, -1e30)
            m_new = jnp.maximum(m, jnp.max(s, axis=0))
            alpha = jnp.exp(m - m_new)
            p = jnp.exp(s - m_new[None])
            l_new = l * alpha + jnp.sum(p, axis=0)
            pv = jnp.sum(jnp.broadcast_to(p, (CK, H, D)) * vc, axis=0)
            acc_new = acc * alpha + pv

            fin = fin_ref[g]

            @pl.when(fin == 1)
            def _():
                comm_ref[0, 0, b] = acc_new
                comm_ref[0, 1, b] = jnp.broadcast_to(m_new, (H, D))
                comm_ref[0, 2, b] = jnp.broadcast_to(l_new, (H, D))

            keep = (fin == 0)
            return (jnp.where(keep, m_new, -1e30),
                    jnp.where(keep, l_new, 0.0),
                    jnp.where(keep, acc_new, 0.0))

        init = (jnp.full((H, 1), -1e30, jnp.float32),
                jnp.zeros((H, 1), jnp.float32),
                jnp.zeros((H, D), jnp.float32))
        lax.fori_loop(0, G, chunk_body, init)

        bsem = pltpu.get_barrier_semaphore()
        for dz in (1, 2, 3):
            pl.semaphore_signal(
                bsem, inc=1,
                device_id=(my_x, my_y, lax.rem(mz + dz, ZDIM)),
                device_id_type=pl.DeviceIdType.MESH,
            )
        pl.semaphore_wait(bsem, 3)

        sends = []
        for dz in (1, 2, 3):
            rdma = pltpu.make_async_remote_copy(
                src_ref=comm_ref.at[0],
                dst_ref=comm_ref.at[ZDIM - dz],
                send_sem=send_sems.at[dz],
                recv_sem=recv_sems.at[ZDIM - dz],
                device_id=(my_x, my_y, lax.rem(mz + dz, ZDIM)),
                device_id_type=pl.DeviceIdType.MESH,
            )
            rdma.start()
            sends.append(rdma)

        for s in (1, 2, 3):
            recv = pltpu.make_async_remote_copy(
                src_ref=comm_ref.at[s],
                dst_ref=comm_ref.at[s],
                send_sem=send_sems.at[0],
                recv_sem=recv_sems.at[s],
                device_id=(my_x, my_y, mz),
                device_id_type=pl.DeviceIdType.MESH,
            )
            recv.wait_recv()
        for rdma in sends:
            rdma.wait_send()

        mall = comm_ref[:, 1]
        mmax = jnp.max(mall, axis=0)
        w = jnp.exp(mall - mmax[None])
        o = jnp.sum(comm_ref[:, 0] * w, axis=0)
        lsum = jnp.sum(comm_ref[:, 2] * w, axis=0)
        out_ref[:, 0] = o / lsum

    return pl.pallas_call(
        body,
        out_shape=jax.ShapeDtypeStruct((B, 1, H, D), jnp.float32),
        in_specs=[
            pl.BlockSpec(memory_space=pltpu.VMEM),
            pl.BlockSpec(memory_space=pl.ANY),
            pl.BlockSpec(memory_space=pl.ANY),
            pl.BlockSpec(memory_space=pltpu.SMEM),
            pl.BlockSpec(memory_space=pltpu.SMEM),
        ],
        out_specs=pl.BlockSpec(memory_space=pltpu.VMEM),
        scratch_shapes=[
            pltpu.VMEM((ZDIM, 3, B, H, D), jnp.float32),
            pltpu.VMEM((2, CH, BS, H, D), jnp.float32),
            pltpu.VMEM((2, CH, BS, H, D), jnp.float32),
            pltpu.SMEM((GMAX,), jnp.int32),
            pltpu.SMEM((GMAX,), jnp.int32),
            pltpu.SMEM((GMAX,), jnp.int32),
            pltpu.SemaphoreType.DMA((2, 2, CH)),
            pltpu.SemaphoreType.DMA((4,)),
            pltpu.SemaphoreType.DMA((4,)),
        ],
        compiler_params=pltpu.CompilerParams(collective_id=0),
    )(Q, K, V, cpt, counts)
